# baseline (speedup 1.0000x reference)
import functools

import jax
import jax.numpy as jnp
from jax import lax
from jax.experimental import pallas as pl
from jax.experimental.pallas import tpu as pltpu
from jax.experimental.pallas import tpu_sc as plsc

_NC = 2
_NS = 16
_NW = _NC * _NS
_K = 64
_GW = 8
_BN = 1024


_NBUF = 2
_NSEG = 2


@functools.lru_cache(maxsize=None)
def _make_sc_scatter_sum(n_pad, ch, c_width):
    mesh = plsc.VectorSubcoreMesh(core_axis_name="c", subcore_axis_name="s")
    cpt = ch // _NW
    seg = cpt // _NSEG
    n_inner = seg // _NBUF
    rows_per_tile = n_pad // _NS
    nfull = rows_per_tile // _K
    nseg16 = c_width // 16

    scratch = [
        pltpu.VMEM_SHARED((n_pad, c_width), jnp.float32),
        pltpu.VMEM((seg, _K), jnp.int32),
        pltpu.VMEM((seg, _K), jnp.int32),
    ]
    scratch += [pltpu.VMEM((_K, c_width), jnp.float32)
                for _ in range(_NBUF)]
    scratch += [pltpu.SemaphoreType.DMA for _ in range(_NBUF)]

    def body(*refs):
        src_r, dst_r, table, out, acc, didx, sidx = refs[:7]
        rows = refs[7:7 + _NBUF]
        gsem = refs[7 + _NBUF:]
        c = lax.axis_index("c")
        s = lax.axis_index("s")
        w = s * _NC + c
        base = pl.multiple_of(s * rows_per_tile, _K)

        def zrow(i, _):
            for k in range(nseg16):
                rows[0][i, pl.ds(k * 16, 16)] = jnp.zeros((16,), jnp.float32)
            return 0
        lax.fori_loop(0, _K, zrow, 0)
        for k in range(nfull):
            pltpu.sync_copy(rows[0], acc.at[pl.ds(base + k * _K, _K)])
        plsc.subcore_barrier()

        lo = w * cpt
        for si in range(_NSEG):
            pltpu.sync_copy(dst_r.at[pl.ds(lo + si * seg, seg)], didx)
            pltpu.sync_copy(src_r.at[pl.ds(lo + si * seg, seg)], sidx)
            for b in range(_NBUF):
                pltpu.async_copy(table.at[sidx.at[b]], rows[b], gsem[b])

            def inner(i, _):
                for b in range(_NBUF):
                    j = i * _NBUF + b
                    pltpu.make_async_copy(table.at[sidx.at[j]], rows[b],
                                          gsem[b]).wait()
                    pltpu.sync_copy(rows[b], acc.at[didx.at[j]], add=True)

                    @pl.when(i < n_inner - 1)
                    def _():
                        pltpu.async_copy(table.at[sidx.at[j + _NBUF]],
                                         rows[b], gsem[b])
                return 0

            lax.fori_loop(0, n_inner, inner, 0)
        plsc.subcore_barrier()

        pltpu.sync_copy(acc.at[pl.ds(base, rows_per_tile)],
                        out.at[c, pl.ds(base, rows_per_tile)])

    return pl.kernel(
        body,
        out_type=jax.ShapeDtypeStruct((_NC, n_pad, c_width), jnp.float32),
        mesh=mesh,
        scratch_types=scratch,
    )


def _tc1_body(degp_ref, x_ref, xs_ref, dinv_ref):
    deg = 1.0 + degp_ref[0][:, 0:1] + degp_ref[1][:, 0:1]
    dinv = lax.rsqrt(deg)
    dinv_ref[...] = dinv
    xs_ref[...] = x_ref[...] * dinv


def _tc2_body(nb, sp_ref, xs_ref, dinv_ref, w_ref, b_ref, rwp_ref, ms_ref):
    dinv = dinv_ref[...]
    t = dinv * (sp_ref[0] + sp_ref[1] + xs_ref[...])
    for i in range(nb):
        h = jnp.maximum(jnp.dot(t, w_ref[i]) + b_ref[i], 0.0)
        p = jnp.dot(h, w_ref[i])
        m = jnp.dot(p, rwp_ref[i])
        ms_ref[:, _GW * i:_GW * (i + 1)] = dinv * m
    ms_ref[:, _GW * nb:] = jnp.zeros((ms_ref.shape[0], ms_ref.shape[1] - _GW * nb),
                                     jnp.float32)


def _tc3_body(dims, s2_ref, ms_ref, dinv_ref, b_ref, rwp_ref, rbp_ref, out_ref):
    u = dinv_ref[...] * (s2_ref[0] + s2_ref[1] + ms_ref[...])
    bn = u.shape[0]
    for i, d in enumerate(dims):
        cvec = jnp.dot(b_ref[i].reshape(1, -1), rwp_ref[i]) + rbp_ref[i]
        z = u[:, _GW * i:_GW * (i + 1)] + cvec
        mask = lax.broadcasted_iota(jnp.int32, (bn, _GW), 1) < d
        z = jnp.where(mask, z, -1e30)
        mx = jnp.max(z, axis=1, keepdims=True)
        e = jnp.exp(z - mx)
        out_ref[:, _GW * i:_GW * (i + 1)] = e / jnp.sum(e, axis=1, keepdims=True)
    pad = out_ref.shape[1] - _GW * len(dims)
    if pad:
        out_ref[:, _GW * len(dims):] = jnp.zeros((bn, pad), jnp.float32)


def kernel(x, edge_index, y, Ws, bs, Rws, Rbs):
    n, h = x.shape
    e = edge_index.shape[1]
    nb = len(Ws)
    dims = [int(rw.shape[1]) for rw in Rws]
    n_pad = -(-n // (_NS * _K * 2)) * (_NS * _K * 2)
    ch = -(-(e // _K) // (_NW * _NBUF * _NSEG)) * (_NW * _NBUF * _NSEG)
    e_pad = ch * _K
    grid = (n_pad // _BN,)
    msc_pad = 128

    x = jnp.pad(x, ((0, n_pad - n), (0, 0)))
    epad = jnp.full((2, e_pad - e), n_pad - 1, jnp.int32)
    ei = jnp.concatenate([edge_index, epad], axis=1)
    src_r = ei[0].reshape(ch, _K)
    dst_r = ei[1].reshape(ch, _K)
    wstack = jnp.stack(Ws)
    bstack = jnp.stack(bs)
    rwp = jnp.stack([jnp.pad(rw, ((0, 0), (0, _GW - rw.shape[1])))
                     for rw in Rws])
    rbp = jnp.stack([jnp.pad(rb, (0, _GW - rb.shape[0])) for rb in Rbs])

    ones_tab = jnp.ones((n_pad, h), jnp.float32)
    zero_idx = jnp.zeros_like(src_r)
    degp = _make_sc_scatter_sum(n_pad, ch, h)(zero_idx, dst_r, ones_tab)

    xs, dinv = pl.pallas_call(
        _tc1_body,
        grid=grid,
        in_specs=[
            pl.BlockSpec((_NC, _BN, h), lambda i: (0, i, 0)),
            pl.BlockSpec((_BN, h), lambda i: (i, 0)),
        ],
        out_specs=[
            pl.BlockSpec((_BN, h), lambda i: (i, 0)),
            pl.BlockSpec((_BN, 1), lambda i: (i, 0)),
        ],
        out_shape=[
            jax.ShapeDtypeStruct((n_pad, h), jnp.float32),
            jax.ShapeDtypeStruct((n_pad, 1), jnp.float32),
        ],
    )(degp, x)

    s1p = _make_sc_scatter_sum(n_pad, ch, h)(src_r, dst_r, xs)

    ms = pl.pallas_call(
        functools.partial(_tc2_body, nb),
        grid=grid,
        in_specs=[
            pl.BlockSpec((_NC, _BN, h), lambda i: (0, i, 0)),
            pl.BlockSpec((_BN, h), lambda i: (i, 0)),
            pl.BlockSpec((_BN, 1), lambda i: (i, 0)),
            pl.BlockSpec((nb, h, h), lambda i: (0, 0, 0)),
            pl.BlockSpec((nb, h), lambda i: (0, 0)),
            pl.BlockSpec((nb, h, _GW), lambda i: (0, 0, 0)),
        ],
        out_specs=pl.BlockSpec((_BN, msc_pad), lambda i: (i, 0)),
        out_shape=jax.ShapeDtypeStruct((n_pad, msc_pad), jnp.float32),
    )(s1p, xs, dinv, wstack, bstack, rwp)

    s2p = _make_sc_scatter_sum(n_pad, ch, msc_pad)(src_r, dst_r, ms)

    out = pl.pallas_call(
        functools.partial(_tc3_body, dims),
        grid=grid,
        in_specs=[
            pl.BlockSpec((_NC, _BN, msc_pad), lambda i: (0, i, 0)),
            pl.BlockSpec((_BN, msc_pad), lambda i: (i, 0)),
            pl.BlockSpec((_BN, 1), lambda i: (i, 0)),
            pl.BlockSpec((nb, h), lambda i: (0, 0)),
            pl.BlockSpec((nb, h, _GW), lambda i: (0, 0, 0)),
            pl.BlockSpec((nb, _GW), lambda i: (0, 0)),
        ],
        out_specs=pl.BlockSpec((_BN, msc_pad), lambda i: (i, 0)),
        out_shape=jax.ShapeDtypeStruct((n_pad, msc_pad), jnp.float32),
    )(s2p, ms, dinv, bstack, rwp, rbp)

    return tuple(out[:n, _GW * i:_GW * i + d] for i, d in enumerate(dims))


# speedup vs baseline: 4.4237x; 4.4237x over previous
import functools

import jax
import jax.numpy as jnp
from jax import lax
from jax.experimental import pallas as pl
from jax.experimental.pallas import tpu as pltpu
from jax.experimental.pallas import tpu_sc as plsc

_NC = 2
_NS = 16
_NW = _NC * _NS
_K = 120
_GW = 8
_BN = 1024


_NBUF = 2
_SEG = 8


def _make_sc_prop(n_pad, ch, c_width):
    mesh = plsc.VectorSubcoreMesh(core_axis_name="c", subcore_axis_name="s")
    cpt = ch // _NW
    nsegs = cpt // _SEG
    seg = _SEG
    n_inner = seg // _NBUF
    rows_per_tile = n_pad // _NS
    nfull = rows_per_tile // _K
    nrem = rows_per_tile % _K
    nseg16 = c_width // 16

    scratch = [
        pltpu.VMEM_SHARED((n_pad, c_width), jnp.float32),
        pltpu.VMEM((seg, _K), jnp.int32),
        pltpu.VMEM((seg, _K), jnp.int32),
    ]
    scratch += [pltpu.VMEM((_K, c_width), jnp.float32)
                for _ in range(_NBUF)]
    scratch += [pltpu.SemaphoreType.DMA for _ in range(_NBUF)]

    def body(*refs):
        src_r, dst_r, table, out, acc, didx, sidx = refs[:7]
        rows = refs[7:7 + _NBUF]
        gsem = refs[7 + _NBUF:]
        c = lax.axis_index("c")
        s = lax.axis_index("s")
        w = s * _NC + c
        base = pl.multiple_of(s * rows_per_tile, 8)

        def zrow(i, _):
            for k in range(nseg16):
                rows[0][i, pl.ds(k * 16, 16)] = jnp.zeros((16,), jnp.float32)
            return 0
        lax.fori_loop(0, _K, zrow, 0)
        for k in range(nfull):
            pltpu.sync_copy(rows[0], acc.at[pl.ds(base + k * _K, _K)])
        if nrem:
            pltpu.sync_copy(rows[0].at[pl.ds(0, nrem)],
                            acc.at[pl.ds(base + nfull * _K, nrem)])
        plsc.subcore_barrier()

        lo = w * cpt
        for si in range(nsegs):
            pltpu.sync_copy(dst_r.at[pl.ds(lo + si * seg, seg)], didx)
            pltpu.sync_copy(src_r.at[pl.ds(lo + si * seg, seg)], sidx)
            for b in range(_NBUF):
                pltpu.async_copy(table.at[sidx.at[b]], rows[b], gsem[b])

            def inner(i, _):
                for b in range(_NBUF):
                    j = i * _NBUF + b
                    pltpu.make_async_copy(table.at[sidx.at[j]], rows[b],
                                          gsem[b]).wait()
                    pltpu.sync_copy(rows[b], acc.at[didx.at[j]], add=True)
                    jn = jnp.minimum(j + _NBUF, seg - 1)
                    pltpu.async_copy(table.at[sidx.at[jn]], rows[b], gsem[b])
                return 0

            lax.fori_loop(0, n_inner, inner, 0)
            for b in range(_NBUF):
                pltpu.make_async_copy(table.at[sidx.at[b]], rows[b],
                                      gsem[b]).wait()
        plsc.subcore_barrier()

        pltpu.sync_copy(acc.at[pl.ds(base, rows_per_tile)],
                        out.at[c, pl.ds(base, rows_per_tile)])

    return pl.kernel(
        body,
        out_type=jax.ShapeDtypeStruct((_NC, n_pad, c_width), jnp.float32),
        mesh=mesh,
        scratch_types=scratch,
    )


def _tc1_body(degp_ref, x_ref, xs_ref, dinv_ref):
    deg = 1.0 + degp_ref[0][:, 0:1] + degp_ref[1][:, 0:1]
    dinv = lax.rsqrt(deg)
    dinv_ref[...] = dinv
    xs_ref[...] = x_ref[...] * dinv


def _tc2_body(nb, sp_ref, xs_ref, dinv_ref, w_ref, b_ref, rwp_ref, ms_ref):
    dinv = dinv_ref[...]
    t = dinv * (sp_ref[0] + sp_ref[1] + xs_ref[...])
    for i in range(nb):
        h = jnp.maximum(jnp.dot(t, w_ref[i]) + b_ref[i], 0.0)
        p = jnp.dot(h, w_ref[i])
        m = jnp.dot(p, rwp_ref[i])
        ms_ref[:, _GW * i:_GW * (i + 1)] = dinv * m
    ms_ref[:, _GW * nb:] = jnp.zeros((ms_ref.shape[0], ms_ref.shape[1] - _GW * nb),
                                     jnp.float32)


def _tc3_body(dims, s2_ref, ms_ref, dinv_ref, b_ref, rwp_ref, rbp_ref, out_ref):
    u = dinv_ref[...] * (s2_ref[0] + s2_ref[1] + ms_ref[...])
    bn = u.shape[0]
    for i, d in enumerate(dims):
        cvec = jnp.dot(b_ref[i].reshape(1, -1), rwp_ref[i]) + rbp_ref[i]
        z = u[:, _GW * i:_GW * (i + 1)] + cvec
        mask = lax.broadcasted_iota(jnp.int32, (bn, _GW), 1) < d
        z = jnp.where(mask, z, -1e30)
        mx = jnp.max(z, axis=1, keepdims=True)
        e = jnp.exp(z - mx)
        out_ref[:, _GW * i:_GW * (i + 1)] = e / jnp.sum(e, axis=1, keepdims=True)
    pad = out_ref.shape[1] - _GW * len(dims)
    if pad:
        out_ref[:, _GW * len(dims):] = jnp.zeros((bn, pad), jnp.float32)


def kernel(x, edge_index, y, Ws, bs, Rws, Rbs):
    n, h = x.shape
    e = edge_index.shape[1]
    nb = len(Ws)
    dims = [int(rw.shape[1]) for rw in Rws]
    n_pad = -(-n // _BN) * _BN
    cpt = (-(-e // _K) + _NW * _SEG - 1) // (_NW * _SEG) * _SEG
    ch = cpt * _NW
    assert _SEG % _NBUF == 0 and n_pad % (_NS * 8) == 0
    e_pad = ch * _K
    grid = (n_pad // _BN,)
    msc_pad = 128

    x = jnp.pad(x, ((0, n_pad - n), (0, 0)))
    epad = jnp.full((2, e_pad - e), n_pad - 1, jnp.int32)
    ei = jnp.concatenate([edge_index, epad], axis=1)
    src_r = ei[0].reshape(ch, _K)
    dst_r = ei[1].reshape(ch, _K)
    wstack = jnp.stack(Ws)
    bstack = jnp.stack(bs)
    rwp = jnp.stack([jnp.pad(rw, ((0, 0), (0, _GW - rw.shape[1])))
                     for rw in Rws])
    rbp = jnp.stack([jnp.pad(rb, (0, _GW - rb.shape[0])) for rb in Rbs])

    ones_tab = jnp.ones((n_pad, h), jnp.float32)
    degp = _make_sc_prop(n_pad, ch, h)(dst_r, dst_r, ones_tab)

    xs, dinv = pl.pallas_call(
        _tc1_body,
        grid=grid,
        in_specs=[
            pl.BlockSpec((_NC, _BN, h), lambda i: (0, i, 0)),
            pl.BlockSpec((_BN, h), lambda i: (i, 0)),
        ],
        out_specs=[
            pl.BlockSpec((_BN, h), lambda i: (i, 0)),
            pl.BlockSpec((_BN, 1), lambda i: (i, 0)),
        ],
        out_shape=[
            jax.ShapeDtypeStruct((n_pad, h), jnp.float32),
            jax.ShapeDtypeStruct((n_pad, 1), jnp.float32),
        ],
    )(degp, x)

    s1p = _make_sc_prop(n_pad, ch, h)(src_r, dst_r, xs)

    ms = pl.pallas_call(
        functools.partial(_tc2_body, nb),
        grid=grid,
        in_specs=[
            pl.BlockSpec((_NC, _BN, h), lambda i: (0, i, 0)),
            pl.BlockSpec((_BN, h), lambda i: (i, 0)),
            pl.BlockSpec((_BN, 1), lambda i: (i, 0)),
            pl.BlockSpec((nb, h, h), lambda i: (0, 0, 0)),
            pl.BlockSpec((nb, h), lambda i: (0, 0)),
            pl.BlockSpec((nb, h, _GW), lambda i: (0, 0, 0)),
        ],
        out_specs=pl.BlockSpec((_BN, msc_pad), lambda i: (i, 0)),
        out_shape=jax.ShapeDtypeStruct((n_pad, msc_pad), jnp.float32),
    )(s1p, xs, dinv, wstack, bstack, rwp)

    s2p = _make_sc_prop(n_pad, ch, msc_pad)(src_r, dst_r, ms)

    out = pl.pallas_call(
        functools.partial(_tc3_body, dims),
        grid=grid,
        in_specs=[
            pl.BlockSpec((_NC, _BN, msc_pad), lambda i: (0, i, 0)),
            pl.BlockSpec((_BN, msc_pad), lambda i: (i, 0)),
            pl.BlockSpec((_BN, 1), lambda i: (i, 0)),
            pl.BlockSpec((nb, h), lambda i: (0, 0)),
            pl.BlockSpec((nb, h, _GW), lambda i: (0, 0, 0)),
            pl.BlockSpec((nb, _GW), lambda i: (0, 0)),
        ],
        out_specs=pl.BlockSpec((_BN, msc_pad), lambda i: (i, 0)),
        out_shape=jax.ShapeDtypeStruct((n_pad, msc_pad), jnp.float32),
    )(s2p, ms, dinv, bstack, rwp, rbp)

    return tuple(out[:n, _GW * i:_GW * i + d] for i, d in enumerate(dims))


# speedup vs baseline: 5.3212x; 1.2029x over previous
import functools

import jax
import jax.numpy as jnp
from jax import lax
from jax.experimental import pallas as pl
from jax.experimental.pallas import tpu as pltpu
from jax.experimental.pallas import tpu_sc as plsc

_NC = 2
_NS = 16
_NW = _NC * _NS
_K = 120
_GW = 8
_BN = 1024


_NBUF = 2
_SEG = 8


def _make_sc_prop(n_pad, ch, c_width):
    mesh = plsc.VectorSubcoreMesh(core_axis_name="c", subcore_axis_name="s")
    cpt = ch // _NW
    nsegs = cpt // _SEG
    seg = _SEG
    n_inner = seg // _NBUF
    rows_per_tile = n_pad // _NS
    nfull = rows_per_tile // _K
    nrem = rows_per_tile % _K
    nseg16 = c_width // 16

    scratch = [
        pltpu.VMEM_SHARED((n_pad, c_width), jnp.float32),
        pltpu.VMEM((seg, _K), jnp.int32),
        pltpu.VMEM((seg, _K), jnp.int32),
    ]
    scratch += [pltpu.VMEM((_K, c_width), jnp.float32)
                for _ in range(_NBUF)]
    scratch += [pltpu.SemaphoreType.DMA for _ in range(_NBUF)]

    def body(*refs):
        src_r, dst_r, table, out, acc, didx, sidx = refs[:7]
        rows = refs[7:7 + _NBUF]
        gsem = refs[7 + _NBUF:]
        c = lax.axis_index("c")
        s = lax.axis_index("s")
        w = s * _NC + c
        base = pl.multiple_of(s * rows_per_tile, 8)

        def zrow(i, _):
            for k in range(nseg16):
                rows[0][i, pl.ds(k * 16, 16)] = jnp.zeros((16,), jnp.float32)
            return 0
        lax.fori_loop(0, _K, zrow, 0)
        for k in range(nfull):
            pltpu.sync_copy(rows[0], acc.at[pl.ds(base + k * _K, _K)])
        if nrem:
            pltpu.sync_copy(rows[0].at[pl.ds(0, nrem)],
                            acc.at[pl.ds(base + nfull * _K, nrem)])
        plsc.subcore_barrier()

        lo = w * cpt
        for si in range(nsegs):
            pltpu.sync_copy(dst_r.at[pl.ds(lo + si * seg, seg)], didx)
            pltpu.sync_copy(src_r.at[pl.ds(lo + si * seg, seg)], sidx)
            def inner(i, _):
                for b in range(_NBUF):
                    j = i * _NBUF + b
                    pltpu.async_copy(table.at[sidx.at[j]], rows[b],
                                     gsem[b]).wait()
                    pltpu.sync_copy(rows[b], acc.at[didx.at[j]], add=True)
                return 0

            lax.fori_loop(0, n_inner, inner, 0)
        plsc.subcore_barrier()

        pltpu.sync_copy(acc.at[pl.ds(base, rows_per_tile)],
                        out.at[c, pl.ds(base, rows_per_tile)])

    return pl.kernel(
        body,
        out_type=jax.ShapeDtypeStruct((_NC, n_pad, c_width), jnp.float32),
        mesh=mesh,
        scratch_types=scratch,
    )


def _tc1_body(degp_ref, x_ref, xs_ref, dinv_ref):
    deg = 1.0 + degp_ref[0][:, 0:1] + degp_ref[1][:, 0:1]
    dinv = lax.rsqrt(deg)
    dinv_ref[...] = dinv
    xs_ref[...] = x_ref[...] * dinv


def _tc2_body(nb, sp_ref, xs_ref, dinv_ref, w_ref, b_ref, rwp_ref, ms_ref):
    dinv = dinv_ref[...]
    t = dinv * (sp_ref[0] + sp_ref[1] + xs_ref[...])
    for i in range(nb):
        h = jnp.maximum(jnp.dot(t, w_ref[i]) + b_ref[i], 0.0)
        p = jnp.dot(h, w_ref[i])
        m = jnp.dot(p, rwp_ref[i])
        ms_ref[:, _GW * i:_GW * (i + 1)] = dinv * m
    ms_ref[:, _GW * nb:] = jnp.zeros((ms_ref.shape[0], ms_ref.shape[1] - _GW * nb),
                                     jnp.float32)


def _tc3_body(dims, s2_ref, ms_ref, dinv_ref, b_ref, rwp_ref, rbp_ref, out_ref):
    u = dinv_ref[...] * (s2_ref[0] + s2_ref[1] + ms_ref[...])
    bn = u.shape[0]
    for i, d in enumerate(dims):
        cvec = jnp.dot(b_ref[i].reshape(1, -1), rwp_ref[i]) + rbp_ref[i]
        z = u[:, _GW * i:_GW * (i + 1)] + cvec
        mask = lax.broadcasted_iota(jnp.int32, (bn, _GW), 1) < d
        z = jnp.where(mask, z, -1e30)
        mx = jnp.max(z, axis=1, keepdims=True)
        e = jnp.exp(z - mx)
        out_ref[:, _GW * i:_GW * (i + 1)] = e / jnp.sum(e, axis=1, keepdims=True)
    pad = out_ref.shape[1] - _GW * len(dims)
    if pad:
        out_ref[:, _GW * len(dims):] = jnp.zeros((bn, pad), jnp.float32)


def kernel(x, edge_index, y, Ws, bs, Rws, Rbs):
    n, h = x.shape
    e = edge_index.shape[1]
    nb = len(Ws)
    dims = [int(rw.shape[1]) for rw in Rws]
    n_pad = -(-n // _BN) * _BN
    cpt = (-(-e // _K) + _NW * _SEG - 1) // (_NW * _SEG) * _SEG
    ch = cpt * _NW
    assert _SEG % _NBUF == 0 and n_pad % (_NS * 8) == 0
    e_pad = ch * _K
    grid = (n_pad // _BN,)
    msc_pad = 128

    x = jnp.pad(x, ((0, n_pad - n), (0, 0)))
    epad = jnp.full((2, e_pad - e), n_pad - 1, jnp.int32)
    ei = jnp.concatenate([edge_index, epad], axis=1)
    src_r = ei[0].reshape(ch, _K)
    dst_r = ei[1].reshape(ch, _K)
    wstack = jnp.stack(Ws)
    bstack = jnp.stack(bs)
    rwp = jnp.stack([jnp.pad(rw, ((0, 0), (0, _GW - rw.shape[1])))
                     for rw in Rws])
    rbp = jnp.stack([jnp.pad(rb, (0, _GW - rb.shape[0])) for rb in Rbs])

    ones_tab = jnp.ones((n_pad, h), jnp.float32)
    degp = _make_sc_prop(n_pad, ch, h)(dst_r, dst_r, ones_tab)

    xs, dinv = pl.pallas_call(
        _tc1_body,
        grid=grid,
        in_specs=[
            pl.BlockSpec((_NC, _BN, h), lambda i: (0, i, 0)),
            pl.BlockSpec((_BN, h), lambda i: (i, 0)),
        ],
        out_specs=[
            pl.BlockSpec((_BN, h), lambda i: (i, 0)),
            pl.BlockSpec((_BN, 1), lambda i: (i, 0)),
        ],
        out_shape=[
            jax.ShapeDtypeStruct((n_pad, h), jnp.float32),
            jax.ShapeDtypeStruct((n_pad, 1), jnp.float32),
        ],
    )(degp, x)

    s1p = _make_sc_prop(n_pad, ch, h)(src_r, dst_r, xs)

    ms = pl.pallas_call(
        functools.partial(_tc2_body, nb),
        grid=grid,
        in_specs=[
            pl.BlockSpec((_NC, _BN, h), lambda i: (0, i, 0)),
            pl.BlockSpec((_BN, h), lambda i: (i, 0)),
            pl.BlockSpec((_BN, 1), lambda i: (i, 0)),
            pl.BlockSpec((nb, h, h), lambda i: (0, 0, 0)),
            pl.BlockSpec((nb, h), lambda i: (0, 0)),
            pl.BlockSpec((nb, h, _GW), lambda i: (0, 0, 0)),
        ],
        out_specs=pl.BlockSpec((_BN, msc_pad), lambda i: (i, 0)),
        out_shape=jax.ShapeDtypeStruct((n_pad, msc_pad), jnp.float32),
    )(s1p, xs, dinv, wstack, bstack, rwp)

    s2p = _make_sc_prop(n_pad, ch, msc_pad)(src_r, dst_r, ms)

    out = pl.pallas_call(
        functools.partial(_tc3_body, dims),
        grid=grid,
        in_specs=[
            pl.BlockSpec((_NC, _BN, msc_pad), lambda i: (0, i, 0)),
            pl.BlockSpec((_BN, msc_pad), lambda i: (i, 0)),
            pl.BlockSpec((_BN, 1), lambda i: (i, 0)),
            pl.BlockSpec((nb, h), lambda i: (0, 0)),
            pl.BlockSpec((nb, h, _GW), lambda i: (0, 0, 0)),
            pl.BlockSpec((nb, _GW), lambda i: (0, 0)),
        ],
        out_specs=pl.BlockSpec((_BN, msc_pad), lambda i: (i, 0)),
        out_shape=jax.ShapeDtypeStruct((n_pad, msc_pad), jnp.float32),
    )(s2p, ms, dinv, bstack, rwp, rbp)

    return tuple(out[:n, _GW * i:_GW * i + d] for i, d in enumerate(dims))


# speedup vs baseline: 11.5300x; 2.1668x over previous
import functools

import jax
import jax.numpy as jnp
from jax import lax
from jax.experimental import pallas as pl
from jax.experimental.pallas import tpu as pltpu
from jax.experimental.pallas import tpu_sc as plsc

_NC = 2
_NS = 16
_NW = _NC * _NS
_K = 128
_GW = 8
_BN = 1024


_NBUF = 1
_SEG = 8


def _make_sc_prop(n_pad, ch, c_width):
    mesh = plsc.VectorSubcoreMesh(core_axis_name="c", subcore_axis_name="s")
    cpt = ch // _NW
    nsegs = cpt // _SEG
    seg = _SEG
    n_inner = seg // _NBUF
    rows_per_tile = n_pad // _NS
    nfull = rows_per_tile // _K
    nrem = rows_per_tile % _K
    nseg16 = c_width // 16

    scratch = [
        pltpu.VMEM_SHARED((n_pad, c_width), jnp.float32),
        pltpu.VMEM((seg, _K), jnp.int32),
        pltpu.VMEM((seg, _K), jnp.int32),
    ]
    scratch += [pltpu.VMEM((_K, c_width), jnp.float32)
                for _ in range(_NBUF)]
    scratch += [pltpu.SemaphoreType.DMA for _ in range(_NBUF)]

    def body(*refs):
        src_r, dst_r, table, out, acc, didx, sidx = refs[:7]
        rows = refs[7:7 + _NBUF]
        gsem = refs[7 + _NBUF:]
        c = lax.axis_index("c")
        s = lax.axis_index("s")
        w = s * _NC + c
        base = pl.multiple_of(s * rows_per_tile, 8)

        def zrow(i, _):
            for k in range(nseg16):
                rows[0][i, pl.ds(k * 16, 16)] = jnp.zeros((16,), jnp.float32)
            return 0
        lax.fori_loop(0, _K, zrow, 0)
        for k in range(nfull):
            pltpu.sync_copy(rows[0], acc.at[pl.ds(base + k * _K, _K)])
        if nrem:
            pltpu.sync_copy(rows[0].at[pl.ds(0, nrem)],
                            acc.at[pl.ds(base + nfull * _K, nrem)])
        plsc.subcore_barrier()

        lo = w * cpt
        for si in range(nsegs):
            pltpu.sync_copy(dst_r.at[pl.ds(lo + si * seg, seg)], didx)
            pltpu.sync_copy(src_r.at[pl.ds(lo + si * seg, seg)], sidx)
            def inner(i, _):
                for b in range(_NBUF):
                    j = i * _NBUF + b
                    pltpu.async_copy(table.at[sidx.at[j]], rows[b],
                                     gsem[b]).wait()
                    pltpu.sync_copy(rows[b], acc.at[didx.at[j]], add=True)
                return 0

            lax.fori_loop(0, n_inner, inner, 0)
        plsc.subcore_barrier()

        pltpu.sync_copy(acc.at[pl.ds(base, rows_per_tile)],
                        out.at[c, pl.ds(base, rows_per_tile)])

    return pl.kernel(
        body,
        out_type=jax.ShapeDtypeStruct((_NC, n_pad, c_width), jnp.float32),
        mesh=mesh,
        scratch_types=scratch,
    )


@functools.lru_cache(maxsize=None)
def _make_sc_degree(n_pad, ch):
    mesh = plsc.VectorSubcoreMesh(core_axis_name="c", subcore_axis_name="s")
    cw = 16
    fire = 4
    cpt = ch // _NW
    nsegs = cpt // _SEG
    n_inner = _SEG // fire
    rows_per_tile = n_pad // _NS
    nfull = rows_per_tile // _K
    nrem = rows_per_tile % _K

    scratch = [
        pltpu.VMEM_SHARED((n_pad, cw), jnp.float32),
        pltpu.VMEM((_K, cw), jnp.float32),
        pltpu.VMEM((_SEG, _K), jnp.int32),
        pltpu.SemaphoreType.DMA,
    ]

    def body(dst_r, out, acc, zbuf, didx, ssem):
        c = lax.axis_index("c")
        s = lax.axis_index("s")
        w = s * _NC + c
        base = pl.multiple_of(s * rows_per_tile, 8)

        def fill(val):
            def row(i, _):
                zbuf[i, pl.ds(0, 16)] = jnp.full((16,), val, jnp.float32)
                return 0
            lax.fori_loop(0, _K, row, 0)

        fill(0.0)
        for k in range(nfull):
            pltpu.sync_copy(zbuf, acc.at[pl.ds(base + k * _K, _K)])
        if nrem:
            pltpu.sync_copy(zbuf.at[pl.ds(0, nrem)],
                            acc.at[pl.ds(base + nfull * _K, nrem)])
        fill(1.0)
        plsc.subcore_barrier()

        lo = w * cpt
        for si in range(nsegs):
            pltpu.sync_copy(dst_r.at[pl.ds(lo + si * _SEG, _SEG)], didx)

            def inner(i, _):
                for b in range(fire):
                    pltpu.async_copy(zbuf, acc.at[didx.at[i * fire + b]],
                                     ssem, add=True)
                for b in range(fire):
                    pltpu.make_async_copy(zbuf, acc.at[didx.at[i * fire + b]],
                                          ssem).wait()
                return 0

            lax.fori_loop(0, n_inner, inner, 0)
        plsc.subcore_barrier()

        pltpu.sync_copy(acc.at[pl.ds(base, rows_per_tile)],
                        out.at[c, pl.ds(base, rows_per_tile)])

    return pl.kernel(
        body,
        out_type=jax.ShapeDtypeStruct((_NC, n_pad, cw), jnp.float32),
        mesh=mesh,
        scratch_types=scratch,
    )


def _tc1_body(degp_ref, x_ref, xs_ref, dinv_ref):
    deg = 1.0 + degp_ref[0][:, 0:1] + degp_ref[1][:, 0:1]
    dinv = lax.rsqrt(deg)
    dinv_ref[...] = dinv
    xs_ref[...] = x_ref[...] * dinv


def _tc2_body(nb, sp_ref, xs_ref, dinv_ref, w_ref, b_ref, rwp_ref, ms_ref):
    dinv = dinv_ref[...]
    t = dinv * (sp_ref[0] + sp_ref[1] + xs_ref[...])
    for i in range(nb):
        h = jnp.maximum(jnp.dot(t, w_ref[i]) + b_ref[i], 0.0)
        p = jnp.dot(h, w_ref[i])
        m = jnp.dot(p, rwp_ref[i])
        ms_ref[:, _GW * i:_GW * (i + 1)] = dinv * m
    ms_ref[:, _GW * nb:] = jnp.zeros((ms_ref.shape[0], ms_ref.shape[1] - _GW * nb),
                                     jnp.float32)


def _tc3_body(dims, s2_ref, ms_ref, dinv_ref, b_ref, rwp_ref, rbp_ref, out_ref):
    u = dinv_ref[...] * (s2_ref[0] + s2_ref[1] + ms_ref[...])
    bn = u.shape[0]
    for i, d in enumerate(dims):
        cvec = jnp.dot(b_ref[i].reshape(1, -1), rwp_ref[i]) + rbp_ref[i]
        z = u[:, _GW * i:_GW * (i + 1)] + cvec
        mask = lax.broadcasted_iota(jnp.int32, (bn, _GW), 1) < d
        z = jnp.where(mask, z, -1e30)
        mx = jnp.max(z, axis=1, keepdims=True)
        e = jnp.exp(z - mx)
        out_ref[:, _GW * i:_GW * (i + 1)] = e / jnp.sum(e, axis=1, keepdims=True)
    pad = out_ref.shape[1] - _GW * len(dims)
    if pad:
        out_ref[:, _GW * len(dims):] = jnp.zeros((bn, pad), jnp.float32)


def kernel(x, edge_index, y, Ws, bs, Rws, Rbs):
    n, h = x.shape
    e = edge_index.shape[1]
    nb = len(Ws)
    dims = [int(rw.shape[1]) for rw in Rws]
    n_pad = -(-n // _BN) * _BN
    cpt = (-(-e // _K) + _NW * _SEG - 1) // (_NW * _SEG) * _SEG
    ch = cpt * _NW
    assert _SEG % _NBUF == 0 and n_pad % (_NS * 8) == 0
    e_pad = ch * _K
    grid = (n_pad // _BN,)
    msc_pad = 128

    x = jnp.pad(x, ((0, n_pad - n), (0, 0)))
    epad = jnp.full((2, e_pad - e), n_pad - 1, jnp.int32)
    ei = jnp.concatenate([edge_index, epad], axis=1)
    src_r = ei[0].reshape(ch, _K)
    dst_r = ei[1].reshape(ch, _K)
    wstack = jnp.stack(Ws)
    bstack = jnp.stack(bs)
    rwp = jnp.stack([jnp.pad(rw, ((0, 0), (0, _GW - rw.shape[1])))
                     for rw in Rws])
    rbp = jnp.stack([jnp.pad(rb, (0, _GW - rb.shape[0])) for rb in Rbs])

    degp = _make_sc_degree(n_pad, ch)(dst_r)

    xs, dinv = pl.pallas_call(
        _tc1_body,
        grid=grid,
        in_specs=[
            pl.BlockSpec((_NC, _BN, 16), lambda i: (0, i, 0)),
            pl.BlockSpec((_BN, h), lambda i: (i, 0)),
        ],
        out_specs=[
            pl.BlockSpec((_BN, h), lambda i: (i, 0)),
            pl.BlockSpec((_BN, 1), lambda i: (i, 0)),
        ],
        out_shape=[
            jax.ShapeDtypeStruct((n_pad, h), jnp.float32),
            jax.ShapeDtypeStruct((n_pad, 1), jnp.float32),
        ],
    )(degp, x)

    s1p = _make_sc_prop(n_pad, ch, h)(src_r, dst_r, xs)

    ms = pl.pallas_call(
        functools.partial(_tc2_body, nb),
        grid=grid,
        in_specs=[
            pl.BlockSpec((_NC, _BN, h), lambda i: (0, i, 0)),
            pl.BlockSpec((_BN, h), lambda i: (i, 0)),
            pl.BlockSpec((_BN, 1), lambda i: (i, 0)),
            pl.BlockSpec((nb, h, h), lambda i: (0, 0, 0)),
            pl.BlockSpec((nb, h), lambda i: (0, 0)),
            pl.BlockSpec((nb, h, _GW), lambda i: (0, 0, 0)),
        ],
        out_specs=pl.BlockSpec((_BN, msc_pad), lambda i: (i, 0)),
        out_shape=jax.ShapeDtypeStruct((n_pad, msc_pad), jnp.float32),
    )(s1p, xs, dinv, wstack, bstack, rwp)

    s2p = _make_sc_prop(n_pad, ch, msc_pad)(src_r, dst_r, ms)

    out = pl.pallas_call(
        functools.partial(_tc3_body, dims),
        grid=grid,
        in_specs=[
            pl.BlockSpec((_NC, _BN, msc_pad), lambda i: (0, i, 0)),
            pl.BlockSpec((_BN, msc_pad), lambda i: (i, 0)),
            pl.BlockSpec((_BN, 1), lambda i: (i, 0)),
            pl.BlockSpec((nb, h), lambda i: (0, 0)),
            pl.BlockSpec((nb, h, _GW), lambda i: (0, 0, 0)),
            pl.BlockSpec((nb, _GW), lambda i: (0, 0)),
        ],
        out_specs=pl.BlockSpec((_BN, msc_pad), lambda i: (i, 0)),
        out_shape=jax.ShapeDtypeStruct((n_pad, msc_pad), jnp.float32),
    )(s2p, ms, dinv, bstack, rwp, rbp)

    return tuple(out[:n, _GW * i:_GW * i + d] for i, d in enumerate(dims))


# speedup vs baseline: 24.5739x; 2.1313x over previous
import functools

import jax
import jax.numpy as jnp
from jax import lax
from jax.experimental import pallas as pl
from jax.experimental.pallas import tpu as pltpu
from jax.experimental.pallas import tpu_sc as plsc

_NC = 2
_NS = 16
_NW = _NC * _NS
_K = 128
_GW = 8
_BN = 1024


_NBUF = 1
_SEG = 8


def _make_sc_prop(n_pad, ch, c_width):
    mesh = plsc.VectorSubcoreMesh(core_axis_name="c", subcore_axis_name="s")
    cpt = ch // _NW
    nsegs = cpt // _SEG
    seg = _SEG
    n_inner = seg // _NBUF
    rows_per_tile = n_pad // _NS
    nfull = rows_per_tile // _K
    nrem = rows_per_tile % _K
    nseg16 = c_width // 16

    scratch = [
        pltpu.VMEM_SHARED((n_pad, c_width), jnp.float32),
        pltpu.VMEM((seg, _K), jnp.int32),
        pltpu.VMEM((seg, _K), jnp.int32),
    ]
    scratch += [pltpu.VMEM((_K, c_width), jnp.float32)
                for _ in range(_NBUF)]
    scratch += [pltpu.SemaphoreType.DMA for _ in range(_NBUF)]

    def body(*refs):
        src_r, dst_r, table, out, acc, didx, sidx = refs[:7]
        rows = refs[7:7 + _NBUF]
        gsem = refs[7 + _NBUF:]
        c = lax.axis_index("c")
        s = lax.axis_index("s")
        w = s * _NC + c
        base = pl.multiple_of(s * rows_per_tile, 8)

        def zrow(i, _):
            for k in range(nseg16):
                rows[0][i, pl.ds(k * 16, 16)] = jnp.zeros((16,), jnp.float32)
            return 0
        lax.fori_loop(0, _K, zrow, 0)
        for k in range(nfull):
            pltpu.sync_copy(rows[0], acc.at[pl.ds(base + k * _K, _K)])
        if nrem:
            pltpu.sync_copy(rows[0].at[pl.ds(0, nrem)],
                            acc.at[pl.ds(base + nfull * _K, nrem)])
        plsc.subcore_barrier()

        lo = w * cpt
        for si in range(nsegs):
            pltpu.sync_copy(dst_r.at[pl.ds(lo + si * seg, seg)], didx)
            pltpu.sync_copy(src_r.at[pl.ds(lo + si * seg, seg)], sidx)
            def inner(i, _):
                for b in range(_NBUF):
                    j = i * _NBUF + b
                    pltpu.async_copy(table.at[sidx.at[j]], rows[b],
                                     gsem[b]).wait()
                    pltpu.sync_copy(rows[b], acc.at[didx.at[j]], add=True)
                return 0

            lax.fori_loop(0, n_inner, inner, 0)
        plsc.subcore_barrier()

        pltpu.sync_copy(acc.at[pl.ds(base, rows_per_tile)],
                        out.at[c, pl.ds(base, rows_per_tile)])

    return pl.kernel(
        body,
        out_type=jax.ShapeDtypeStruct((_NC, n_pad, c_width), jnp.float32),
        mesh=mesh,
        scratch_types=scratch,
    )


@functools.lru_cache(maxsize=None)
def _make_sc_degree(n_pad, ch):
    mesh = plsc.VectorSubcoreMesh(core_axis_name="c", subcore_axis_name="s")
    cw = 16
    fire = 4
    cpt = ch // _NW
    nsegs = cpt // _SEG
    n_inner = _SEG // fire
    rows_per_tile = n_pad // _NS
    nfull = rows_per_tile // _K
    nrem = rows_per_tile % _K

    scratch = [
        pltpu.VMEM_SHARED((n_pad, cw), jnp.float32),
        pltpu.VMEM((_K, cw), jnp.float32),
        pltpu.VMEM((_SEG, _K), jnp.int32),
        pltpu.SemaphoreType.DMA,
    ]

    def body(dst_r, out, acc, zbuf, didx, ssem):
        c = lax.axis_index("c")
        s = lax.axis_index("s")
        w = s * _NC + c
        base = pl.multiple_of(s * rows_per_tile, 8)

        def fill(val):
            def row(i, _):
                zbuf[i, pl.ds(0, 16)] = jnp.full((16,), val, jnp.float32)
                return 0
            lax.fori_loop(0, _K, row, 0)

        fill(0.0)
        for k in range(nfull):
            pltpu.sync_copy(zbuf, acc.at[pl.ds(base + k * _K, _K)])
        if nrem:
            pltpu.sync_copy(zbuf.at[pl.ds(0, nrem)],
                            acc.at[pl.ds(base + nfull * _K, nrem)])
        fill(1.0)
        plsc.subcore_barrier()

        lo = w * cpt
        for si in range(nsegs):
            pltpu.sync_copy(dst_r.at[pl.ds(lo + si * _SEG, _SEG)], didx)

            def inner(i, _):
                for b in range(fire):
                    pltpu.async_copy(zbuf, acc.at[didx.at[i * fire + b]],
                                     ssem, add=True).wait()
                return 0

            lax.fori_loop(0, n_inner, inner, 0)
        plsc.subcore_barrier()

        pltpu.sync_copy(acc.at[pl.ds(base, rows_per_tile)],
                        out.at[c, pl.ds(base, rows_per_tile)])

    return pl.kernel(
        body,
        out_type=jax.ShapeDtypeStruct((_NC, n_pad, cw), jnp.float32),
        mesh=mesh,
        scratch_types=scratch,
    )


def _tc1_body(degp_ref, x_ref, xs_ref, dinv_ref):
    deg = 1.0 + degp_ref[0][:, 0:1] + degp_ref[1][:, 0:1]
    dinv = lax.rsqrt(deg)
    dinv_ref[...] = dinv
    xs_ref[...] = x_ref[...] * dinv


def _tc2_body(nb, sp_ref, xs_ref, dinv_ref, w_ref, b_ref, rwp_ref, ms_ref):
    dinv = dinv_ref[...]
    t = dinv * (sp_ref[0] + sp_ref[1] + xs_ref[...])
    for i in range(nb):
        h = jnp.maximum(jnp.dot(t, w_ref[i]) + b_ref[i], 0.0)
        p = jnp.dot(h, w_ref[i])
        m = jnp.dot(p, rwp_ref[i])
        ms_ref[:, _GW * i:_GW * (i + 1)] = dinv * m
    ms_ref[:, _GW * nb:] = jnp.zeros((ms_ref.shape[0], ms_ref.shape[1] - _GW * nb),
                                     jnp.float32)


def _tc3_body(dims, s2_ref, ms_ref, dinv_ref, b_ref, rwp_ref, rbp_ref, out_ref):
    u = dinv_ref[...] * (s2_ref[0] + s2_ref[1] + ms_ref[...])
    bn = u.shape[0]
    for i, d in enumerate(dims):
        cvec = jnp.dot(b_ref[i].reshape(1, -1), rwp_ref[i]) + rbp_ref[i]
        z = u[:, _GW * i:_GW * (i + 1)] + cvec
        mask = lax.broadcasted_iota(jnp.int32, (bn, _GW), 1) < d
        z = jnp.where(mask, z, -1e30)
        mx = jnp.max(z, axis=1, keepdims=True)
        e = jnp.exp(z - mx)
        out_ref[:, _GW * i:_GW * (i + 1)] = e / jnp.sum(e, axis=1, keepdims=True)
    pad = out_ref.shape[1] - _GW * len(dims)
    if pad:
        out_ref[:, _GW * len(dims):] = jnp.zeros((bn, pad), jnp.float32)


def kernel(x, edge_index, y, Ws, bs, Rws, Rbs):
    n, h = x.shape
    e = edge_index.shape[1]
    nb = len(Ws)
    dims = [int(rw.shape[1]) for rw in Rws]
    n_pad = -(-n // _BN) * _BN
    cpt = (-(-e // _K) + _NW * _SEG - 1) // (_NW * _SEG) * _SEG
    ch = cpt * _NW
    assert _SEG % _NBUF == 0 and n_pad % (_NS * 8) == 0
    e_pad = ch * _K
    grid = (n_pad // _BN,)
    msc_pad = 128

    x = jnp.pad(x, ((0, n_pad - n), (0, 0)))
    spread = n + jnp.arange(e_pad - e, dtype=jnp.int32) % (n_pad - n)
    ei = jnp.concatenate([edge_index, jnp.stack([spread, spread])], axis=1)
    src_r = ei[0].reshape(ch, _K)
    dst_r = ei[1].reshape(ch, _K)
    wstack = jnp.stack(Ws)
    bstack = jnp.stack(bs)
    rwp = jnp.stack([jnp.pad(rw, ((0, 0), (0, _GW - rw.shape[1])))
                     for rw in Rws])
    rbp = jnp.stack([jnp.pad(rb, (0, _GW - rb.shape[0])) for rb in Rbs])

    degp = _make_sc_degree(n_pad, ch)(dst_r)

    xs, dinv = pl.pallas_call(
        _tc1_body,
        grid=grid,
        in_specs=[
            pl.BlockSpec((_NC, _BN, 16), lambda i: (0, i, 0)),
            pl.BlockSpec((_BN, h), lambda i: (i, 0)),
        ],
        out_specs=[
            pl.BlockSpec((_BN, h), lambda i: (i, 0)),
            pl.BlockSpec((_BN, 1), lambda i: (i, 0)),
        ],
        out_shape=[
            jax.ShapeDtypeStruct((n_pad, h), jnp.float32),
            jax.ShapeDtypeStruct((n_pad, 1), jnp.float32),
        ],
    )(degp, x)

    s1p = _make_sc_prop(n_pad, ch, h)(src_r, dst_r, xs)

    ms = pl.pallas_call(
        functools.partial(_tc2_body, nb),
        grid=grid,
        in_specs=[
            pl.BlockSpec((_NC, _BN, h), lambda i: (0, i, 0)),
            pl.BlockSpec((_BN, h), lambda i: (i, 0)),
            pl.BlockSpec((_BN, 1), lambda i: (i, 0)),
            pl.BlockSpec((nb, h, h), lambda i: (0, 0, 0)),
            pl.BlockSpec((nb, h), lambda i: (0, 0)),
            pl.BlockSpec((nb, h, _GW), lambda i: (0, 0, 0)),
        ],
        out_specs=pl.BlockSpec((_BN, msc_pad), lambda i: (i, 0)),
        out_shape=jax.ShapeDtypeStruct((n_pad, msc_pad), jnp.float32),
    )(s1p, xs, dinv, wstack, bstack, rwp)

    s2p = _make_sc_prop(n_pad, ch, msc_pad)(src_r, dst_r, ms)

    out = pl.pallas_call(
        functools.partial(_tc3_body, dims),
        grid=grid,
        in_specs=[
            pl.BlockSpec((_NC, _BN, msc_pad), lambda i: (0, i, 0)),
            pl.BlockSpec((_BN, msc_pad), lambda i: (i, 0)),
            pl.BlockSpec((_BN, 1), lambda i: (i, 0)),
            pl.BlockSpec((nb, h), lambda i: (0, 0)),
            pl.BlockSpec((nb, h, _GW), lambda i: (0, 0, 0)),
            pl.BlockSpec((nb, _GW), lambda i: (0, 0)),
        ],
        out_specs=pl.BlockSpec((_BN, msc_pad), lambda i: (i, 0)),
        out_shape=jax.ShapeDtypeStruct((n_pad, msc_pad), jnp.float32),
    )(s2p, ms, dinv, bstack, rwp, rbp)

    return tuple(out[:n, _GW * i:_GW * i + d] for i, d in enumerate(dims))


# speedup vs baseline: 27.6176x; 1.1239x over previous
import functools

import jax
import jax.numpy as jnp
from jax import lax
from jax.experimental import pallas as pl
from jax.experimental.pallas import tpu as pltpu
from jax.experimental.pallas import tpu_sc as plsc

_NC = 2
_NS = 16
_NW = _NC * _NS
_K = 128
_GW = 8
_BN = 632


_NBUF = 2
_SEG = 8


def _make_sc_prop(n_pad, ch, c_width):
    mesh = plsc.VectorSubcoreMesh(core_axis_name="c", subcore_axis_name="s")
    cpt = ch // _NW
    nsegs = cpt // _SEG
    seg = _SEG
    n_inner = seg // _NBUF
    rows_per_tile = n_pad // _NS
    nfull = rows_per_tile // _K
    nrem = rows_per_tile % _K
    nseg16 = c_width // 16

    scratch = [
        pltpu.VMEM_SHARED((n_pad, c_width), jnp.float32),
        pltpu.VMEM((seg, _K), jnp.int32),
        pltpu.VMEM((seg, _K), jnp.int32),
    ]
    scratch += [pltpu.VMEM((_K, c_width), jnp.float32)
                for _ in range(_NBUF)]
    scratch += [pltpu.SemaphoreType.DMA for _ in range(_NBUF)]

    def body(*refs):
        src_r, dst_r, table, out, acc, didx, sidx = refs[:7]
        rows = refs[7:7 + _NBUF]
        gsem = refs[7 + _NBUF:]
        c = lax.axis_index("c")
        s = lax.axis_index("s")
        w = s * _NC + c
        base = pl.multiple_of(s * rows_per_tile, 8)

        def zrow(i, _):
            for k in range(nseg16):
                rows[0][i, pl.ds(k * 16, 16)] = jnp.zeros((16,), jnp.float32)
            return 0
        lax.fori_loop(0, _K, zrow, 0)
        for k in range(nfull):
            pltpu.sync_copy(rows[0], acc.at[pl.ds(base + k * _K, _K)])
        if nrem:
            pltpu.sync_copy(rows[0].at[pl.ds(0, nrem)],
                            acc.at[pl.ds(base + nfull * _K, nrem)])
        plsc.subcore_barrier()

        lo = w * cpt
        for si in range(nsegs):
            pltpu.sync_copy(dst_r.at[pl.ds(lo + si * seg, seg)], didx)
            pltpu.sync_copy(src_r.at[pl.ds(lo + si * seg, seg)], sidx)
            for b in range(_NBUF):
                pltpu.async_copy(table.at[sidx.at[b]], rows[b], gsem[b])

            def inner(i, _):
                for b in range(_NBUF):
                    j = i * _NBUF + b
                    pltpu.make_async_copy(table.at[sidx.at[j]], rows[b],
                                          gsem[b]).wait()
                    pltpu.sync_copy(rows[b], acc.at[didx.at[j]], add=True)
                    jn = jnp.minimum(j + _NBUF, seg - 1)
                    pltpu.async_copy(table.at[sidx.at[jn]], rows[b], gsem[b])
                return 0

            lax.fori_loop(0, n_inner, inner, 0)
            for b in range(_NBUF):
                pltpu.make_async_copy(table.at[sidx.at[b]], rows[b],
                                      gsem[b]).wait()
        plsc.subcore_barrier()

        pltpu.sync_copy(acc.at[pl.ds(base, rows_per_tile)],
                        out.at[c, pl.ds(base, rows_per_tile)])

    return pl.kernel(
        body,
        out_type=jax.ShapeDtypeStruct((_NC, n_pad, c_width), jnp.float32),
        mesh=mesh,
        scratch_types=scratch,
    )


@functools.lru_cache(maxsize=None)
def _make_sc_degree(n_pad, ch):
    mesh = plsc.VectorSubcoreMesh(core_axis_name="c", subcore_axis_name="s")
    cw = 16
    fire = 4
    cpt = ch // _NW
    nsegs = cpt // _SEG
    n_inner = _SEG // fire
    rows_per_tile = n_pad // _NS
    nfull = rows_per_tile // _K
    nrem = rows_per_tile % _K

    scratch = [
        pltpu.VMEM_SHARED((n_pad, cw), jnp.float32),
        pltpu.VMEM((_K, cw), jnp.float32),
        pltpu.VMEM((_SEG, _K), jnp.int32),
        pltpu.SemaphoreType.DMA,
    ]

    def body(dst_r, out, acc, zbuf, didx, ssem):
        c = lax.axis_index("c")
        s = lax.axis_index("s")
        w = s * _NC + c
        base = pl.multiple_of(s * rows_per_tile, 8)

        def fill(val):
            def row(i, _):
                zbuf[i, pl.ds(0, 16)] = jnp.full((16,), val, jnp.float32)
                return 0
            lax.fori_loop(0, _K, row, 0)

        fill(0.0)
        for k in range(nfull):
            pltpu.sync_copy(zbuf, acc.at[pl.ds(base + k * _K, _K)])
        if nrem:
            pltpu.sync_copy(zbuf.at[pl.ds(0, nrem)],
                            acc.at[pl.ds(base + nfull * _K, nrem)])
        fill(1.0)
        plsc.subcore_barrier()

        lo = w * cpt
        for si in range(nsegs):
            pltpu.sync_copy(dst_r.at[pl.ds(lo + si * _SEG, _SEG)], didx)

            def inner(i, _):
                for b in range(fire):
                    pltpu.async_copy(zbuf, acc.at[didx.at[i * fire + b]],
                                     ssem, add=True).wait()
                return 0

            lax.fori_loop(0, n_inner, inner, 0)
        plsc.subcore_barrier()

        pltpu.sync_copy(acc.at[pl.ds(base, rows_per_tile)],
                        out.at[c, pl.ds(base, rows_per_tile)])

    return pl.kernel(
        body,
        out_type=jax.ShapeDtypeStruct((_NC, n_pad, cw), jnp.float32),
        mesh=mesh,
        scratch_types=scratch,
    )


def _tc1_body(degp_ref, x_ref, xs_ref, dinv_ref):
    deg = 1.0 + degp_ref[0][:, 0:1] + degp_ref[1][:, 0:1]
    dinv = lax.rsqrt(deg)
    dinv_ref[...] = dinv
    xs_ref[...] = x_ref[...] * dinv


def _tc2_body(nb, sp_ref, xs_ref, dinv_ref, w_ref, b_ref, rwp_ref, ms_ref):
    dinv = dinv_ref[...]
    t = dinv * (sp_ref[0] + sp_ref[1] + xs_ref[...])
    for i in range(nb):
        h = jnp.maximum(jnp.dot(t, w_ref[i]) + b_ref[i], 0.0)
        p = jnp.dot(h, w_ref[i])
        m = jnp.dot(p, rwp_ref[i])
        ms_ref[:, _GW * i:_GW * (i + 1)] = dinv * m
    ms_ref[:, _GW * nb:] = jnp.zeros((ms_ref.shape[0], ms_ref.shape[1] - _GW * nb),
                                     jnp.float32)


def _tc3_body(dims, s2_ref, ms_ref, dinv_ref, b_ref, rwp_ref, rbp_ref, out_ref):
    u = dinv_ref[...] * (s2_ref[0] + s2_ref[1] + ms_ref[...])
    bn = u.shape[0]
    for i, d in enumerate(dims):
        cvec = jnp.dot(b_ref[i].reshape(1, -1), rwp_ref[i]) + rbp_ref[i]
        z = u[:, _GW * i:_GW * (i + 1)] + cvec
        mask = lax.broadcasted_iota(jnp.int32, (bn, _GW), 1) < d
        z = jnp.where(mask, z, -1e30)
        mx = jnp.max(z, axis=1, keepdims=True)
        e = jnp.exp(z - mx)
        out_ref[:, _GW * i:_GW * (i + 1)] = e / jnp.sum(e, axis=1, keepdims=True)
    pad = out_ref.shape[1] - _GW * len(dims)
    if pad:
        out_ref[:, _GW * len(dims):] = jnp.zeros((bn, pad), jnp.float32)


def kernel(x, edge_index, y, Ws, bs, Rws, Rbs):
    n, h = x.shape
    e = edge_index.shape[1]
    nb = len(Ws)
    dims = [int(rw.shape[1]) for rw in Rws]
    n_pad = -(-n // _BN) * _BN
    cpt = (-(-e // _K) + _NW * _SEG - 1) // (_NW * _SEG) * _SEG
    ch = cpt * _NW
    assert _SEG % _NBUF == 0 and n_pad % (_NS * 8) == 0
    e_pad = ch * _K
    grid = (n_pad // _BN,)
    msc_pad = 128

    x = jnp.pad(x, ((0, n_pad - n), (0, 0)))
    spread = n + jnp.arange(e_pad - e, dtype=jnp.int32) % (n_pad - n)
    ei = jnp.concatenate([edge_index, jnp.stack([spread, spread])], axis=1)
    src_r = ei[0].reshape(ch, _K)
    dst_r = ei[1].reshape(ch, _K)
    wstack = jnp.stack(Ws)
    bstack = jnp.stack(bs)
    rwp = jnp.stack([jnp.pad(rw, ((0, 0), (0, _GW - rw.shape[1])))
                     for rw in Rws])
    rbp = jnp.stack([jnp.pad(rb, (0, _GW - rb.shape[0])) for rb in Rbs])

    degp = _make_sc_degree(n_pad, ch)(dst_r)

    xs, dinv = pl.pallas_call(
        _tc1_body,
        grid=grid,
        in_specs=[
            pl.BlockSpec((_NC, _BN, 16), lambda i: (0, i, 0)),
            pl.BlockSpec((_BN, h), lambda i: (i, 0)),
        ],
        out_specs=[
            pl.BlockSpec((_BN, h), lambda i: (i, 0)),
            pl.BlockSpec((_BN, 1), lambda i: (i, 0)),
        ],
        out_shape=[
            jax.ShapeDtypeStruct((n_pad, h), jnp.float32),
            jax.ShapeDtypeStruct((n_pad, 1), jnp.float32),
        ],
    )(degp, x)

    s1p = _make_sc_prop(n_pad, ch, h)(src_r, dst_r, xs)

    ms = pl.pallas_call(
        functools.partial(_tc2_body, nb),
        grid=grid,
        in_specs=[
            pl.BlockSpec((_NC, _BN, h), lambda i: (0, i, 0)),
            pl.BlockSpec((_BN, h), lambda i: (i, 0)),
            pl.BlockSpec((_BN, 1), lambda i: (i, 0)),
            pl.BlockSpec((nb, h, h), lambda i: (0, 0, 0)),
            pl.BlockSpec((nb, h), lambda i: (0, 0)),
            pl.BlockSpec((nb, h, _GW), lambda i: (0, 0, 0)),
        ],
        out_specs=pl.BlockSpec((_BN, msc_pad), lambda i: (i, 0)),
        out_shape=jax.ShapeDtypeStruct((n_pad, msc_pad), jnp.float32),
    )(s1p, xs, dinv, wstack, bstack, rwp)

    s2p = _make_sc_prop(n_pad, ch, msc_pad)(src_r, dst_r, ms)

    out = pl.pallas_call(
        functools.partial(_tc3_body, dims),
        grid=grid,
        in_specs=[
            pl.BlockSpec((_NC, _BN, msc_pad), lambda i: (0, i, 0)),
            pl.BlockSpec((_BN, msc_pad), lambda i: (i, 0)),
            pl.BlockSpec((_BN, 1), lambda i: (i, 0)),
            pl.BlockSpec((nb, h), lambda i: (0, 0)),
            pl.BlockSpec((nb, h, _GW), lambda i: (0, 0, 0)),
            pl.BlockSpec((nb, _GW), lambda i: (0, 0)),
        ],
        out_specs=pl.BlockSpec((_BN, msc_pad), lambda i: (i, 0)),
        out_shape=jax.ShapeDtypeStruct((n_pad, msc_pad), jnp.float32),
    )(s2p, ms, dinv, bstack, rwp, rbp)

    return tuple(out[:n, _GW * i:_GW * i + d] for i, d in enumerate(dims))


# speedup vs baseline: 29.2738x; 1.0600x over previous
import functools

import jax
import jax.numpy as jnp
from jax import lax
from jax.experimental import pallas as pl
from jax.experimental.pallas import tpu as pltpu
from jax.experimental.pallas import tpu_sc as plsc

_NC = 2
_NS = 16
_NW = _NC * _NS
_K = 128
_GW = 8
_BN = 632


_NBUF = 2
_SEG = 8


def _make_sc_prop(n_pad, ch, c_width):
    mesh = plsc.VectorSubcoreMesh(core_axis_name="c", subcore_axis_name="s")
    cpt = ch // _NW
    nsegs = cpt // _SEG
    seg = _SEG
    n_inner = seg // _NBUF
    rows_per_tile = n_pad // _NS
    nfull = rows_per_tile // _K
    nrem = rows_per_tile % _K
    nseg16 = c_width // 16

    scratch = [
        pltpu.VMEM_SHARED((n_pad, c_width), jnp.float32),
        pltpu.VMEM((seg, _K), jnp.int32),
        pltpu.VMEM((seg, _K), jnp.int32),
    ]
    scratch += [pltpu.VMEM((_K, c_width), jnp.float32)
                for _ in range(_NBUF)]
    scratch += [pltpu.SemaphoreType.DMA for _ in range(_NBUF)]

    def body(*refs):
        src_r, dst_r, table, out, acc, didx, sidx = refs[:7]
        rows = refs[7:7 + _NBUF]
        gsem = refs[7 + _NBUF:]
        c = lax.axis_index("c")
        s = lax.axis_index("s")
        w = s * _NC + c
        base = pl.multiple_of(s * rows_per_tile, 8)

        def zrow(i, _):
            for k in range(nseg16):
                rows[0][i, pl.ds(k * 16, 16)] = jnp.zeros((16,), jnp.float32)
            return 0
        lax.fori_loop(0, _K, zrow, 0)
        for k in range(nfull):
            pltpu.sync_copy(rows[0], acc.at[pl.ds(base + k * _K, _K)])
        if nrem:
            pltpu.sync_copy(rows[0].at[pl.ds(0, nrem)],
                            acc.at[pl.ds(base + nfull * _K, nrem)])
        plsc.subcore_barrier()

        lo = w * cpt
        for si in range(nsegs):
            pltpu.sync_copy(dst_r.at[pl.ds(lo + si * seg, seg)], didx)
            pltpu.sync_copy(src_r.at[pl.ds(lo + si * seg, seg)], sidx)
            for b in range(_NBUF):
                pltpu.async_copy(table.at[sidx.at[b]], rows[b], gsem[b])

            def inner(i, _):
                for b in range(_NBUF):
                    j = i * _NBUF + b
                    pltpu.make_async_copy(table.at[sidx.at[j]], rows[b],
                                          gsem[b]).wait()
                    pltpu.sync_copy(rows[b], acc.at[didx.at[j]], add=True)

                    @pl.when(i < n_inner - 1)
                    def _():
                        pltpu.async_copy(table.at[sidx.at[j + _NBUF]],
                                         rows[b], gsem[b])
                return 0

            lax.fori_loop(0, n_inner, inner, 0)
        plsc.subcore_barrier()

        pltpu.sync_copy(acc.at[pl.ds(base, rows_per_tile)],
                        out.at[c, pl.ds(base, rows_per_tile)])

    return pl.kernel(
        body,
        out_type=jax.ShapeDtypeStruct((_NC, n_pad, c_width), jnp.float32),
        mesh=mesh,
        scratch_types=scratch,
    )


@functools.lru_cache(maxsize=None)
def _make_sc_degree(n_pad, ch):
    mesh = plsc.VectorSubcoreMesh(core_axis_name="c", subcore_axis_name="s")
    cw = 16
    fire = 4
    cpt = ch // _NW
    nsegs = cpt // _SEG
    n_inner = _SEG // fire
    rows_per_tile = n_pad // _NS
    nfull = rows_per_tile // _K
    nrem = rows_per_tile % _K

    scratch = [
        pltpu.VMEM_SHARED((n_pad, cw), jnp.float32),
        pltpu.VMEM((_K, cw), jnp.float32),
        pltpu.VMEM((_SEG, _K), jnp.int32),
        pltpu.SemaphoreType.DMA,
    ]

    def body(dst_r, out, acc, zbuf, didx, ssem):
        c = lax.axis_index("c")
        s = lax.axis_index("s")
        w = s * _NC + c
        base = pl.multiple_of(s * rows_per_tile, 8)

        def fill(val):
            def row(i, _):
                zbuf[i, pl.ds(0, 16)] = jnp.full((16,), val, jnp.float32)
                return 0
            lax.fori_loop(0, _K, row, 0)

        fill(0.0)
        for k in range(nfull):
            pltpu.sync_copy(zbuf, acc.at[pl.ds(base + k * _K, _K)])
        if nrem:
            pltpu.sync_copy(zbuf.at[pl.ds(0, nrem)],
                            acc.at[pl.ds(base + nfull * _K, nrem)])
        fill(1.0)
        plsc.subcore_barrier()

        lo = w * cpt
        for si in range(nsegs):
            pltpu.sync_copy(dst_r.at[pl.ds(lo + si * _SEG, _SEG)], didx)

            def inner(i, _):
                for b in range(fire):
                    pltpu.async_copy(zbuf, acc.at[didx.at[i * fire + b]],
                                     ssem, add=True).wait()
                return 0

            lax.fori_loop(0, n_inner, inner, 0)
        plsc.subcore_barrier()

        pltpu.sync_copy(acc.at[pl.ds(base, rows_per_tile)],
                        out.at[c, pl.ds(base, rows_per_tile)])

    return pl.kernel(
        body,
        out_type=jax.ShapeDtypeStruct((_NC, n_pad, cw), jnp.float32),
        mesh=mesh,
        scratch_types=scratch,
    )


def _tc1_body(degp_ref, x_ref, xs_ref, dinv_ref):
    deg = 1.0 + degp_ref[0][:, 0:1] + degp_ref[1][:, 0:1]
    dinv = lax.rsqrt(deg)
    dinv_ref[...] = dinv
    xs_ref[...] = x_ref[...] * dinv


def _tc2_body(nb, sp_ref, xs_ref, dinv_ref, w_ref, b_ref, rwp_ref, ms_ref):
    dinv = dinv_ref[...]
    t = dinv * (sp_ref[0] + sp_ref[1] + xs_ref[...])
    for i in range(nb):
        h = jnp.maximum(jnp.dot(t, w_ref[i]) + b_ref[i], 0.0)
        p = jnp.dot(h, w_ref[i])
        m = jnp.dot(p, rwp_ref[i])
        ms_ref[:, _GW * i:_GW * (i + 1)] = dinv * m
    ms_ref[:, _GW * nb:] = jnp.zeros((ms_ref.shape[0], ms_ref.shape[1] - _GW * nb),
                                     jnp.float32)


def _tc3_body(dims, s2_ref, ms_ref, dinv_ref, b_ref, rwp_ref, rbp_ref, out_ref):
    u = dinv_ref[...] * (s2_ref[0] + s2_ref[1] + ms_ref[...])
    bn = u.shape[0]
    for i, d in enumerate(dims):
        cvec = jnp.dot(b_ref[i].reshape(1, -1), rwp_ref[i]) + rbp_ref[i]
        z = u[:, _GW * i:_GW * (i + 1)] + cvec
        mask = lax.broadcasted_iota(jnp.int32, (bn, _GW), 1) < d
        z = jnp.where(mask, z, -1e30)
        mx = jnp.max(z, axis=1, keepdims=True)
        e = jnp.exp(z - mx)
        out_ref[:, _GW * i:_GW * (i + 1)] = e / jnp.sum(e, axis=1, keepdims=True)
    pad = out_ref.shape[1] - _GW * len(dims)
    if pad:
        out_ref[:, _GW * len(dims):] = jnp.zeros((bn, pad), jnp.float32)


def kernel(x, edge_index, y, Ws, bs, Rws, Rbs):
    n, h = x.shape
    e = edge_index.shape[1]
    nb = len(Ws)
    dims = [int(rw.shape[1]) for rw in Rws]
    n_pad = -(-n // _BN) * _BN
    cpt = (-(-e // _K) + _NW * _SEG - 1) // (_NW * _SEG) * _SEG
    ch = cpt * _NW
    assert _SEG % _NBUF == 0 and n_pad % (_NS * 8) == 0
    e_pad = ch * _K
    grid = (n_pad // _BN,)
    msc_pad = 128

    x = jnp.pad(x, ((0, n_pad - n), (0, 0)))
    spread = n + jnp.arange(e_pad - e, dtype=jnp.int32) % (n_pad - n)
    ei = jnp.concatenate([edge_index, jnp.stack([spread, spread])], axis=1)
    src_r = ei[0].reshape(ch, _K)
    dst_r = ei[1].reshape(ch, _K)
    wstack = jnp.stack(Ws)
    bstack = jnp.stack(bs)
    rwp = jnp.stack([jnp.pad(rw, ((0, 0), (0, _GW - rw.shape[1])))
                     for rw in Rws])
    rbp = jnp.stack([jnp.pad(rb, (0, _GW - rb.shape[0])) for rb in Rbs])

    degp = _make_sc_degree(n_pad, ch)(dst_r)

    xs, dinv = pl.pallas_call(
        _tc1_body,
        grid=grid,
        in_specs=[
            pl.BlockSpec((_NC, _BN, 16), lambda i: (0, i, 0)),
            pl.BlockSpec((_BN, h), lambda i: (i, 0)),
        ],
        out_specs=[
            pl.BlockSpec((_BN, h), lambda i: (i, 0)),
            pl.BlockSpec((_BN, 1), lambda i: (i, 0)),
        ],
        out_shape=[
            jax.ShapeDtypeStruct((n_pad, h), jnp.float32),
            jax.ShapeDtypeStruct((n_pad, 1), jnp.float32),
        ],
    )(degp, x)

    s1p = _make_sc_prop(n_pad, ch, h)(src_r, dst_r, xs)

    ms = pl.pallas_call(
        functools.partial(_tc2_body, nb),
        grid=grid,
        in_specs=[
            pl.BlockSpec((_NC, _BN, h), lambda i: (0, i, 0)),
            pl.BlockSpec((_BN, h), lambda i: (i, 0)),
            pl.BlockSpec((_BN, 1), lambda i: (i, 0)),
            pl.BlockSpec((nb, h, h), lambda i: (0, 0, 0)),
            pl.BlockSpec((nb, h), lambda i: (0, 0)),
            pl.BlockSpec((nb, h, _GW), lambda i: (0, 0, 0)),
        ],
        out_specs=pl.BlockSpec((_BN, msc_pad), lambda i: (i, 0)),
        out_shape=jax.ShapeDtypeStruct((n_pad, msc_pad), jnp.float32),
    )(s1p, xs, dinv, wstack, bstack, rwp)

    s2p = _make_sc_prop(n_pad, ch, msc_pad)(src_r, dst_r, ms)

    out = pl.pallas_call(
        functools.partial(_tc3_body, dims),
        grid=grid,
        in_specs=[
            pl.BlockSpec((_NC, _BN, msc_pad), lambda i: (0, i, 0)),
            pl.BlockSpec((_BN, msc_pad), lambda i: (i, 0)),
            pl.BlockSpec((_BN, 1), lambda i: (i, 0)),
            pl.BlockSpec((nb, h), lambda i: (0, 0)),
            pl.BlockSpec((nb, h, _GW), lambda i: (0, 0, 0)),
            pl.BlockSpec((nb, _GW), lambda i: (0, 0)),
        ],
        out_specs=pl.BlockSpec((_BN, msc_pad), lambda i: (i, 0)),
        out_shape=jax.ShapeDtypeStruct((n_pad, msc_pad), jnp.float32),
    )(s2p, ms, dinv, bstack, rwp, rbp)

    return tuple(out[:n, _GW * i:_GW * i + d] for i, d in enumerate(dims))
